# baseline (speedup 1.0000x reference)
import functools

import jax
import jax.numpy as jnp
from jax import lax
from jax.experimental import pallas as pl
from jax.experimental.pallas import tpu as pltpu
from jax.experimental.pallas import tpu_sc as plsc

_LEVELS = 16
_DIM = 128
_BATCH = 16384
_LANES = 16
_IDX_CHUNK = 128


@functools.cache
def _build(batch, dim, levels):
    info = plsc.get_sparse_core_info()
    num_workers = info.num_cores * info.num_subcores
    b_per_w = batch // num_workers
    n_chunks = b_per_w // _IDX_CHUNK
    mesh = plsc.VectorSubcoreMesh(core_axis_name="c", subcore_axis_name="s")

    @functools.partial(
        pl.kernel,
        mesh=mesh,
        out_type=jax.ShapeDtypeStruct((batch, dim), jnp.float32),
        scratch_types=[
            pltpu.VMEM((b_per_w,), jnp.float32),
            pltpu.VMEM((n_chunks, _IDX_CHUNK), jnp.int32),
            pltpu.VMEM((b_per_w, dim), jnp.float32),
            pltpu.VMEM_SHARED((levels, dim), jnp.float32),
            pltpu.SemaphoreType.DMA,
            pltpu.SemaphoreType.DMA,
            pltpu.SemaphoreType.DMA,
            pltpu.SemaphoreType.DMA,
        ],
    )
    def k(sev_hbm, table_hbm, out_hbm, sev_v, idx_v, rows_v, table_s,
          sem_t, sem_ga, sem_gb, sem_o):
        sem_g = [sem_ga, sem_gb]
        sid = lax.axis_index("s")
        wid = sid * info.num_cores + lax.axis_index("c")
        base = wid * b_per_w

        staging = []

        @pl.when(sid == 0)
        def _():
            staging.append(pltpu.async_copy(table_hbm, table_s, sem_t))

        pltpu.sync_copy(sev_hbm.at[pl.ds(base, b_per_w)], sev_v)
        scale = jnp.float32(levels - 1)
        hi = jnp.int32(levels - 1)
        lo = jnp.int32(0)
        for j in range(n_chunks):
            for i in range(_IDX_CHUNK // _LANES):
                s = sev_v[pl.ds(j * _IDX_CHUNK + i * _LANES, _LANES)]
                q = (s * scale).astype(jnp.int32)
                q = jnp.minimum(jnp.maximum(q, lo), hi)
                idx_v[j, pl.ds(i * _LANES, _LANES)] = q

        @pl.when(sid == 0)
        def _():
            staging[0].wait()

        plsc.subcore_barrier()

        def gather(j):
            return pltpu.async_copy(
                table_s.at[idx_v.at[j]],
                rows_v.at[pl.ds(j * _IDX_CHUNK, _IDX_CHUNK)],
                sem_g[j % 2],
            )

        gathers = [gather(0), gather(1)]
        writes = []
        for j in range(n_chunks):
            gathers[j].wait()
            if j + 2 < n_chunks:
                gathers.append(gather(j + 2))
            writes.append(
                pltpu.async_copy(
                    rows_v.at[pl.ds(j * _IDX_CHUNK, _IDX_CHUNK)],
                    out_hbm.at[pl.ds(base + j * _IDX_CHUNK, _IDX_CHUNK)],
                    sem_o,
                )
            )
        for w in writes:
            w.wait()

    return k


def kernel(severity, table):
    levels, dim = table.shape
    return _build(severity.shape[0], dim, levels)(severity, table)


# speedup vs baseline: 1.0095x; 1.0095x over previous
import functools

import jax
import jax.numpy as jnp
from jax import lax
from jax.experimental import pallas as pl
from jax.experimental.pallas import tpu as pltpu
from jax.experimental.pallas import tpu_sc as plsc

_LANES = 16


@functools.cache
def _build(batch, dim, levels):
    info = plsc.get_sparse_core_info()
    num_workers = info.num_cores * info.num_subcores
    b_per_w = batch // num_workers

    chunks = []
    if b_per_w % 128 == 0 and b_per_w >= 256:
        n_mid = (b_per_w - 128) // 128
        chunks.append(("s", 0, 0, 64))
        for m in range(n_mid):
            chunks.append(("m", m, 64 + m * 128, 128))
        chunks.append(("s", 1, b_per_w - 64, 64))
        n_small, n_mid_rows = 2, n_mid
    else:
        n_mid_rows = b_per_w // 128
        for m in range(n_mid_rows):
            chunks.append(("m", m, m * 128, 128))
        n_small = 2
    n_chunks = len(chunks)
    mesh = plsc.VectorSubcoreMesh(core_axis_name="c", subcore_axis_name="s")

    @functools.partial(
        pl.kernel,
        mesh=mesh,
        out_type=jax.ShapeDtypeStruct((batch, dim), jnp.float32),
        scratch_types=[
            pltpu.VMEM((b_per_w,), jnp.float32),
            pltpu.VMEM((n_small, 64), jnp.int32),
            pltpu.VMEM((n_mid_rows, 128), jnp.int32),
            pltpu.VMEM((b_per_w, dim), jnp.float32),
            pltpu.VMEM_SHARED((levels, dim), jnp.float32),
            pltpu.SemaphoreType.DMA,
            pltpu.SemaphoreType.DMA((n_chunks,)),
            pltpu.SemaphoreType.DMA,
        ],
    )
    def k(sev_hbm, table_hbm, out_hbm, sev_v, idx_s, idx_m, rows_v, table_s,
          sem_t, sem_g, sem_o):
        sid = lax.axis_index("s")
        wid = sid * info.num_cores + lax.axis_index("c")
        base = wid * b_per_w

        staging = []

        @pl.when(sid == 0)
        def _():
            staging.append(pltpu.async_copy(table_hbm, table_s, sem_t))

        pltpu.sync_copy(sev_hbm.at[pl.ds(base, b_per_w)], sev_v)
        scale = jnp.float32(levels - 1)
        hi = jnp.int32(levels - 1)
        lo = jnp.int32(0)
        for kind, row, gstart, size in chunks:
            ref = idx_s if kind == "s" else idx_m
            for i in range(size // _LANES):
                s = sev_v[pl.ds(gstart + i * _LANES, _LANES)]
                q = (s * scale).astype(jnp.int32)
                q = jnp.minimum(jnp.maximum(q, lo), hi)
                ref[row, pl.ds(i * _LANES, _LANES)] = q

        @pl.when(sid == 0)
        def _():
            staging[0].wait()

        plsc.subcore_barrier()
        gathers = []
        for ci, (kind, row, gstart, size) in enumerate(chunks):
            ref = idx_s if kind == "s" else idx_m
            gathers.append(
                pltpu.async_copy(
                    table_s.at[ref.at[row]],
                    rows_v.at[pl.ds(gstart, size)],
                    sem_g.at[ci],
                )
            )
        writes = []
        for ci, (kind, row, gstart, size) in enumerate(chunks):
            gathers[ci].wait()
            writes.append(
                pltpu.async_copy(
                    rows_v.at[pl.ds(gstart, size)],
                    out_hbm.at[pl.ds(base + gstart, size)],
                    sem_o,
                )
            )
        for w in writes:
            w.wait()

    return k


def kernel(severity, table):
    levels, dim = table.shape
    return _build(severity.shape[0], dim, levels)(severity, table)


# speedup vs baseline: 1.0115x; 1.0020x over previous
import functools

import jax
import jax.numpy as jnp
from jax import lax
from jax.experimental import pallas as pl
from jax.experimental.pallas import tpu as pltpu
from jax.experimental.pallas import tpu_sc as plsc

_LANES = 16


@functools.cache
def _build(batch, dim, levels):
    info = plsc.get_sparse_core_info()
    num_workers = info.num_cores * info.num_subcores
    b_per_w = batch // num_workers

    chunks = []
    if b_per_w % 128 == 0 and b_per_w >= 256:
        n_mid = (b_per_w - 128) // 128
        chunks.append(("s", 0, 0, 64))
        for m in range(n_mid):
            chunks.append(("m", m, 64 + m * 128, 128))
        chunks.append(("s", 1, b_per_w - 64, 64))
        n_small, n_mid_rows = 2, n_mid
    else:
        n_mid_rows = b_per_w // 128
        for m in range(n_mid_rows):
            chunks.append(("m", m, m * 128, 128))
        n_small = 2
    n_chunks = len(chunks)
    mesh = plsc.VectorSubcoreMesh(core_axis_name="c", subcore_axis_name="s")

    @functools.partial(
        pl.kernel,
        mesh=mesh,
        out_type=jax.ShapeDtypeStruct((batch, dim), jnp.float32),
        scratch_types=[
            pltpu.VMEM((b_per_w,), jnp.float32),
            pltpu.VMEM((b_per_w,), jnp.int32),
            pltpu.VMEM((b_per_w, dim), jnp.float32),
            pltpu.VMEM_SHARED((levels, dim), jnp.float32),
            pltpu.SemaphoreType.DMA,
            pltpu.SemaphoreType.DMA((n_chunks,)),
            pltpu.SemaphoreType.DMA,
        ],
    )
    def k(sev_hbm, table_hbm, out_hbm, sev_v, idx_f, rows_v, table_s,
          sem_t, sem_g, sem_o):
        sid = lax.axis_index("s")
        wid = sid * info.num_cores + lax.axis_index("c")
        base = wid * b_per_w

        staging = []

        @pl.when(sid == 0)
        def _():
            staging.append(pltpu.async_copy(table_hbm, table_s, sem_t))

        pltpu.sync_copy(sev_hbm.at[pl.ds(base, b_per_w)], sev_v)
        scale = jnp.float32(levels - 1)
        hi = jnp.int32(levels - 1)
        lo = jnp.int32(0)

        def quant_group(g, _):
            s = sev_v[pl.ds(g * _LANES, _LANES)]
            q = (s * scale).astype(jnp.int32)
            q = jnp.minimum(jnp.maximum(q, lo), hi)
            idx_f[pl.ds(g * _LANES, _LANES)] = q
            return _

        lax.fori_loop(0, b_per_w // _LANES, quant_group, 0)

        @pl.when(sid == 0)
        def _():
            staging[0].wait()

        plsc.subcore_barrier()
        gathers = []
        for ci, (kind, row, gstart, size) in enumerate(chunks):
            gathers.append(
                pltpu.async_copy(
                    table_s.at[idx_f.at[pl.ds(gstart, size)]],
                    rows_v.at[pl.ds(gstart, size)],
                    sem_g.at[ci],
                )
            )
        writes = []
        for ci, (kind, row, gstart, size) in enumerate(chunks):
            gathers[ci].wait()
            writes.append(
                pltpu.async_copy(
                    rows_v.at[pl.ds(gstart, size)],
                    out_hbm.at[pl.ds(base + gstart, size)],
                    sem_o,
                )
            )
        for w in writes:
            w.wait()

    return k


def kernel(severity, table):
    levels, dim = table.shape
    return _build(severity.shape[0], dim, levels)(severity, table)


# speedup vs baseline: 1.0139x; 1.0024x over previous
import functools

import jax
import jax.numpy as jnp
from jax import lax
from jax.experimental import pallas as pl
from jax.experimental.pallas import tpu as pltpu
from jax.experimental.pallas import tpu_sc as plsc

_LANES = 16


@functools.cache
def _build(batch, dim, levels):
    info = plsc.get_sparse_core_info()
    num_workers = info.num_cores * info.num_subcores
    b_per_w = batch // num_workers

    if b_per_w % 128 == 0 and b_per_w >= 256:
        n_mid = (b_per_w - 128) // 128
        chunks = ([(0, 64)]
                  + [(64 + m * 128, 128) for m in range(n_mid)]
                  + [(b_per_w - 64, 64)])
    else:
        chunks = [(m * 128, 128) for m in range(b_per_w // 128)]
    n_chunks = len(chunks)
    mesh = plsc.VectorSubcoreMesh(core_axis_name="c", subcore_axis_name="s")

    @functools.partial(
        pl.kernel,
        mesh=mesh,
        out_type=jax.ShapeDtypeStruct((batch, dim), jnp.float32),
        scratch_types=[
            pltpu.VMEM((b_per_w,), jnp.float32),
            pltpu.VMEM((b_per_w,), jnp.int32),
            pltpu.VMEM((b_per_w, dim), jnp.float32),
            pltpu.VMEM_SHARED((levels, dim), jnp.float32),
            pltpu.SemaphoreType.DMA,
            pltpu.SemaphoreType.DMA((n_chunks,)),
            pltpu.SemaphoreType.DMA,
        ],
    )
    def k(sev_hbm, table_hbm, out_hbm, sev_v, idx_f, rows_v, table_s,
          sem_t, sem_g, sem_o):
        sid = lax.axis_index("s")
        wid = sid * info.num_cores + lax.axis_index("c")
        base = wid * b_per_w

        staging = []

        @pl.when(sid == 0)
        def _():
            staging.append(pltpu.async_copy(table_hbm, table_s, sem_t))

        pltpu.sync_copy(sev_hbm.at[pl.ds(base, b_per_w)], sev_v)
        scale = jnp.float32(levels - 1)
        hi = jnp.int32(levels - 1)
        lo = jnp.int32(0)

        def quant_group(g, _):
            s = sev_v[pl.ds(g * _LANES, _LANES)]
            q = (s * scale).astype(jnp.int32)
            q = jnp.minimum(jnp.maximum(q, lo), hi)
            idx_f[pl.ds(g * _LANES, _LANES)] = q
            return _

        lax.fori_loop(0, b_per_w // _LANES, quant_group, 0)

        @pl.when(sid == 0)
        def _():
            staging[0].wait()

        plsc.subcore_barrier()
        gathers = []
        for ci, (gstart, size) in enumerate(chunks):
            gathers.append(
                pltpu.async_copy(
                    table_s.at[idx_f.at[pl.ds(gstart, size)]],
                    rows_v.at[pl.ds(gstart, size)],
                    sem_g.at[ci],
                )
            )
        writes = []
        for ci, (gstart, size) in enumerate(chunks):
            gathers[ci].wait()
            writes.append(
                pltpu.async_copy(
                    rows_v.at[pl.ds(gstart, size)],
                    out_hbm.at[pl.ds(base + gstart, size)],
                    sem_o,
                )
            )
        for w in writes:
            w.wait()

    return k


def kernel(severity, table):
    levels, dim = table.shape
    return _build(severity.shape[0], dim, levels)(severity, table)
